# Initial kernel scaffold; baseline (speedup 1.0000x reference)
import jax
import jax.numpy as jnp
from jax.experimental import pallas as pl


def kernel(input_ids, labels, weight):
    raise NotImplementedError("write your pallas kernel here")


# speedup vs baseline: 1.5709x; 1.5709x over previous
import functools
import math

import jax
import jax.numpy as jnp
from jax import lax
from jax.experimental import pallas as pl
from jax.experimental.pallas import tpu as pltpu
from jax.experimental.pallas import tpu_sc as plsc

D_MODEL = 2048
HEAD_DIM = 128
ROPE_THETA = 10000.0

NC, NS = 2, 16
NW = NC * NS
CHUNK = 16


def _gather_body(chunks_per_worker, idx_hbm, tbl_hbm, out_hbm,
                 idx_v, bufs, sem0, sem1):
    wid = lax.axis_index("s") * NC + lax.axis_index("c")
    rows_per_worker = chunks_per_worker * CHUNK
    base = wid * rows_per_worker
    pltpu.sync_copy(idx_hbm.at[wid], idx_v)
    sems = (sem0, sem1)

    pltpu.async_copy(tbl_hbm.at[idx_v.at[0]], bufs.at[0], sems[0])
    pltpu.async_copy(tbl_hbm.at[idx_v.at[1]], bufs.at[1], sems[1])

    @pl.loop(0, chunks_per_worker - 2, step=2)
    def _(g):
        for b in range(2):
            j = g + b
            pltpu.make_async_copy(tbl_hbm.at[idx_v.at[j]], bufs.at[b],
                                  sems[b]).wait()
            pltpu.sync_copy(bufs.at[b],
                            out_hbm.at[pl.ds(base + j * CHUNK, CHUNK)])
            pltpu.async_copy(tbl_hbm.at[idx_v.at[j + 2]], bufs.at[b], sems[b])

    for b in range(2):
        j = chunks_per_worker - 2 + b
        pltpu.make_async_copy(tbl_hbm.at[idx_v.at[j]], bufs.at[b],
                              sems[b]).wait()
        pltpu.sync_copy(bufs.at[b],
                        out_hbm.at[pl.ds(base + j * CHUNK, CHUNK)])


def _emb_gather(ids_flat, weight):
    n_tok = ids_flat.shape[0]
    chunks_per_worker = n_tok // (NW * CHUNK)
    ids3 = ids_flat.reshape(NW, chunks_per_worker, CHUNK)
    mesh = plsc.VectorSubcoreMesh(core_axis_name="c", subcore_axis_name="s")
    k = pl.kernel(
        functools.partial(_gather_body, chunks_per_worker),
        out_type=jax.ShapeDtypeStruct((n_tok, D_MODEL), jnp.float32),
        mesh=mesh,
        scratch_types=[
            pltpu.VMEM((chunks_per_worker, CHUNK), jnp.int32),
            pltpu.VMEM((2, CHUNK, D_MODEL), jnp.float32),
            pltpu.SemaphoreType.DMA,
            pltpu.SemaphoreType.DMA,
        ],
    )
    return k(ids3, weight)


def _mask_body(block_rows, seqlen, o_ref):
    i = pl.program_id(0)
    r = lax.broadcasted_iota(jnp.int32, (block_rows, seqlen), 0) + i * block_rows
    c = lax.broadcasted_iota(jnp.int32, (block_rows, seqlen), 1)
    o_ref[...] = jnp.where(c > r, float("-inf"), 0.0).astype(jnp.float32)


def _causal_mask(seqlen):
    block_rows = 256
    return pl.pallas_call(
        functools.partial(_mask_body, block_rows, seqlen),
        out_shape=jax.ShapeDtypeStruct((seqlen, seqlen), jnp.float32),
        grid=(seqlen // block_rows,),
        out_specs=pl.BlockSpec((block_rows, seqlen), lambda i: (i, 0)),
    )()


def _freqs_body(end, half, cos_ref, sin_ref):
    t = lax.broadcasted_iota(jnp.int32, (end, half), 0).astype(jnp.float32)
    k = lax.broadcasted_iota(jnp.int32, (end, half), 1).astype(jnp.float32)
    inv = jnp.exp(k * (-2.0 * math.log(ROPE_THETA) / HEAD_DIM))
    ang = t * inv
    cos_ref[...] = jnp.cos(ang)
    sin_ref[...] = jnp.sin(ang)


def _freqs_cis(end):
    half = HEAD_DIM // 2
    cos, sin = pl.pallas_call(
        functools.partial(_freqs_body, end, half),
        out_shape=[jax.ShapeDtypeStruct((end, half), jnp.float32)] * 2,
    )()
    return jax.lax.complex(cos, sin)


def kernel(input_ids, labels, weight):
    bsz, seqlen = input_ids.shape
    flat = _emb_gather(input_ids.reshape(bsz * seqlen), weight)
    hidden = flat.reshape(bsz, seqlen, D_MODEL)
    mask = _causal_mask(seqlen)
    freqs = _freqs_cis(4096)
    return (hidden, freqs, mask, labels)


# speedup vs baseline: 1.5722x; 1.0008x over previous
import functools
import math

import jax
import jax.numpy as jnp
from jax import lax
from jax.experimental import pallas as pl
from jax.experimental.pallas import tpu as pltpu
from jax.experimental.pallas import tpu_sc as plsc

D_MODEL = 2048
HEAD_DIM = 128
ROPE_THETA = 10000.0

NC, NS = 2, 16
NW = NC * NS
CHUNK = 8
NBUF = 4


def _gather_body(chunks_per_worker, idx_hbm, tbl_hbm, out_hbm,
                 idx_v, bufs, gs0, gs1, gs2, gs3, os0, os1, os2, os3):
    wid = lax.axis_index("s") * NC + lax.axis_index("c")
    base = wid * chunks_per_worker * CHUNK
    pltpu.sync_copy(idx_hbm.at[wid], idx_v)
    gsems = (gs0, gs1, gs2, gs3)
    osems = (os0, os1, os2, os3)
    CH = chunks_per_worker

    def start_gather(j, b):
        pltpu.async_copy(tbl_hbm.at[idx_v.at[j]], bufs.at[b], gsems[b])

    def wait_gather(j, b):
        pltpu.make_async_copy(tbl_hbm.at[idx_v.at[j]], bufs.at[b],
                              gsems[b]).wait()

    def start_out(j, b):
        pltpu.async_copy(bufs.at[b],
                         out_hbm.at[pl.ds(base + j * CHUNK, CHUNK)], osems[b])

    def wait_out(j, b):
        pltpu.make_async_copy(bufs.at[b],
                              out_hbm.at[pl.ds(base + j * CHUNK, CHUNK)],
                              osems[b]).wait()

    start_gather(0, 0)
    start_gather(1, 1)
    for j in (0, 1):
        wait_gather(j, j)
        start_out(j, j)
        start_gather(j + 2, j + 2)

    @pl.loop(2, CH - 2, step=NBUF)
    def _(g):
        for db in range(NBUF):
            j = g + db
            b_cur = (2 + db) % NBUF
            b_nxt = db % NBUF
            wait_gather(j, b_cur)
            start_out(j, b_cur)
            wait_out(j - 2, b_nxt)
            start_gather(j + 2, b_nxt)

    for j in (CH - 2, CH - 1):
        b_cur = j % NBUF
        wait_gather(j, b_cur)
        start_out(j, b_cur)
        wait_out(j - 2, (j - 2) % NBUF)
    for j in (CH - 2, CH - 1):
        wait_out(j, j % NBUF)


def _emb_gather(ids_flat, weight):
    n_tok = ids_flat.shape[0]
    chunks_per_worker = n_tok // (NW * CHUNK)
    ids3 = ids_flat.reshape(NW, chunks_per_worker, CHUNK)
    mesh = plsc.VectorSubcoreMesh(core_axis_name="c", subcore_axis_name="s")
    k = pl.kernel(
        functools.partial(_gather_body, chunks_per_worker),
        out_type=jax.ShapeDtypeStruct((n_tok, D_MODEL), jnp.float32),
        mesh=mesh,
        scratch_types=[
            pltpu.VMEM((chunks_per_worker, CHUNK), jnp.int32),
            pltpu.VMEM((NBUF, CHUNK, D_MODEL), jnp.float32),
        ] + [pltpu.SemaphoreType.DMA] * (2 * NBUF),
    )
    return k(ids3, weight)


def _mask_body(block_rows, seqlen, o_ref):
    i = pl.program_id(0)
    r = lax.broadcasted_iota(jnp.int32, (block_rows, seqlen), 0) + i * block_rows
    c = lax.broadcasted_iota(jnp.int32, (block_rows, seqlen), 1)
    o_ref[...] = jnp.where(c > r, float("-inf"), 0.0).astype(jnp.float32)


def _causal_mask(seqlen):
    block_rows = 256
    return pl.pallas_call(
        functools.partial(_mask_body, block_rows, seqlen),
        out_shape=jax.ShapeDtypeStruct((seqlen, seqlen), jnp.float32),
        grid=(seqlen // block_rows,),
        out_specs=pl.BlockSpec((block_rows, seqlen), lambda i: (i, 0)),
    )()


def _freqs_body(end, half, cos_ref, sin_ref):
    t = lax.broadcasted_iota(jnp.int32, (end, half), 0).astype(jnp.float32)
    k = lax.broadcasted_iota(jnp.int32, (end, half), 1).astype(jnp.float32)
    inv = jnp.exp(k * (-2.0 * math.log(ROPE_THETA) / HEAD_DIM))
    ang = t * inv
    cos_ref[...] = jnp.cos(ang)
    sin_ref[...] = jnp.sin(ang)


def _freqs_cis(end):
    half = HEAD_DIM // 2
    cos, sin = pl.pallas_call(
        functools.partial(_freqs_body, end, half),
        out_shape=[jax.ShapeDtypeStruct((end, half), jnp.float32)] * 2,
    )()
    return jax.lax.complex(cos, sin)


def kernel(input_ids, labels, weight):
    bsz, seqlen = input_ids.shape
    flat = _emb_gather(input_ids.reshape(bsz * seqlen), weight)
    hidden = flat.reshape(bsz, seqlen, D_MODEL)
    mask = _causal_mask(seqlen)
    freqs = _freqs_cis(4096)
    return (hidden, freqs, mask, labels)
